# baseline (speedup 1.0000x reference)
import functools

import jax
import jax.numpy as jnp
from jax import lax
from jax.experimental import pallas as pl
from jax.experimental.pallas import tpu as pltpu
from jax.experimental.pallas import tpu_sc as plsc

_BATCH = 16384
_EMBED = 64
_NUM_CORES = 2
_NUM_SUBCORES = 16
_NW = _NUM_CORES * _NUM_SUBCORES
_BPW = _BATCH // _NW
_CHUNK = 128
_NCHUNK = _BPW // _CHUNK

_mesh = plsc.VectorSubcoreMesh(core_axis_name="c", subcore_axis_name="s")


@functools.partial(
    pl.kernel,
    mesh=_mesh,
    compiler_params=pltpu.CompilerParams(
        needs_layout_passes=False, use_tc_tiling_on_sc=False),
    out_type=jax.ShapeDtypeStruct((_BATCH,), jnp.float32),
    scratch_types=[
        pltpu.VMEM((_NCHUNK, _CHUNK), jnp.int32),
        pltpu.VMEM((_NCHUNK, _CHUNK), jnp.int32),
        pltpu.VMEM((_BPW, _EMBED), jnp.float32),
        pltpu.VMEM((_BPW, _EMBED), jnp.float32),
        pltpu.VMEM((_BPW,), jnp.float32),
        pltpu.SemaphoreType.DMA,
    ],
)
def _sc_kernel(uid_hbm, iid_hbm, ut_hbm, it_hbm, out_hbm,
               uid_v, iid_v, urows, irows, out_v, sem):
    wid = lax.axis_index("s") * _NUM_CORES + lax.axis_index("c")
    base = wid * _BPW

    for j in range(_NCHUNK):
        pltpu.sync_copy(uid_hbm.at[pl.ds(base + j * _CHUNK, _CHUNK)],
                        uid_v.at[j])
        pltpu.sync_copy(iid_hbm.at[pl.ds(base + j * _CHUNK, _CHUNK)],
                        iid_v.at[j])

    copies = []
    for j in range(_NCHUNK):
        copies.append(pltpu.async_copy(
            ut_hbm.at[uid_v.at[j]], urows.at[pl.ds(j * _CHUNK, _CHUNK)], sem))
        copies.append(pltpu.async_copy(
            it_hbm.at[iid_v.at[j]], irows.at[pl.ds(j * _CHUNK, _CHUNK)], sem))
    for c in copies:
        c.wait()

    lane = lax.iota(jnp.int32, 16)

    def group_body(g, carry):
        acc = jnp.zeros((16,), jnp.float32)
        for r in range(16):
            row = g * 16 + r
            p = jnp.zeros((16,), jnp.float32)
            for c in range(_EMBED // 16):
                u = urows[row, pl.ds(c * 16, 16)]
                v = irows[row, pl.ds(c * 16, 16)]
                p = p + u * v
            acc = jnp.where(lane == r, jnp.sum(p), acc)
        out_v[pl.ds(g * 16, 16)] = acc
        return carry

    lax.fori_loop(0, _BPW // 16, group_body, 0)

    pltpu.sync_copy(out_v, out_hbm.at[pl.ds(base, _BPW)])


def kernel(user_ids, item_ids, user_table, item_table):
    return _sc_kernel(user_ids, item_ids, user_table, item_table)
